# baseline (speedup 1.0000x reference)
import functools

import jax
import jax.numpy as jnp
from jax import lax
from jax.experimental import pallas as pl
from jax.experimental.pallas import tpu as pltpu
from jax.experimental.pallas import tpu_sc as plsc

_GAMMA = 12.0
_GAMMA_TYPE = 6.0
_HID = 64
_TDIM = 32
_C = 64
_NIDX = 6
_WB = 400
_NB = 200


@functools.cache
def _build_relayout(N, ntab, width):
  k = 128 // width
  out_rows = N // k
  grid0 = out_rows // (_WB if width == _HID else _NB)
  blk = out_rows // grid0
  f32 = jnp.float32

  def body(*refs):
    ins = refs[:ntab]
    outs = refs[ntab:]
    j = pl.program_id(1)
    for jj in range(k):
      @pl.when(j == jj)
      def _(jj=jj):
        for a, o in zip(ins, outs):
          o[:, jj * width:(jj + 1) * width] = a[...]

  return pl.pallas_call(
      body,
      grid=(grid0, k),
      in_specs=[pl.BlockSpec((blk, width), lambda i, j: (i + j * grid0, 0))
                ] * ntab,
      out_specs=[pl.BlockSpec((blk, 128), lambda i, j: (i, 0))] * ntab,
      out_shape=[jax.ShapeDtypeStruct((out_rows, 128), f32)] * ntab,
      compiler_params=pltpu.CompilerParams(
          dimension_semantics=("arbitrary", "arbitrary")),
  )


@functools.cache
def _build_sc(B, D, gamma):
  info = plsc.get_sparse_core_info()
  NC, NS, L = info.num_cores, info.num_subcores, info.num_lanes
  NW = NC * NS
  Q = D // L
  assert B % (NW * _C) == 0
  per_w = B // NW
  n_chunks = per_w // _C
  assert n_chunks % 2 == 0
  groups = _C // L
  f32 = jnp.float32
  i32 = jnp.int32
  mesh = plsc.VectorSubcoreMesh(core_axis_name="c", subcore_axis_name="s")

  def _set():
    return [
        pltpu.VMEM((_NIDX * _C,), i32),
        pltpu.VMEM((2 * _C, 128), f32),
        pltpu.VMEM((_C, 128), f32),
        pltpu.VMEM((_C, 128), f32),
        pltpu.SemaphoreType.DMA,
    ]

  @functools.partial(
      pl.kernel,
      mesh=mesh,
      compiler_params=pltpu.CompilerParams(
          needs_layout_passes=False,
          disable_bounds_checks=True,
      ),
      out_type=jax.ShapeDtypeStruct((B,), f32),
      scratch_types=_set() + _set() + [pltpu.VMEM((per_w,), f32)],
  )
  def kge(pack_hbm, a_hbm, r_hbm, v_hbm, out_hbm, *scratch):
    set0 = scratch[0:5]
    set1 = scratch[5:10]
    sc_v = scratch[10]
    wid = lax.axis_index("s") * NC + lax.axis_index("c")
    base = wid * per_w

    def copies(bufs):
      idx_v, a_v, r_v, v_v, sem = bufs
      ht = idx_v.at[pl.ds(0, 2 * _C)]
      rr = idx_v.at[pl.ds(2 * _C, _C)]
      return [
          pltpu.make_async_copy(a_hbm.at[ht], a_v, sem),
          pltpu.make_async_copy(r_hbm.at[rr], r_v, sem),
          pltpu.make_async_copy(v_hbm.at[rr], v_v, sem),
      ]

    def start_chunk(bufs, ci):
      idx_v = bufs[0]
      gchunk = wid * n_chunks + ci
      pltpu.sync_copy(pack_hbm.at[pl.ds(gchunk * (_NIDX * _C), _NIDX * _C)],
                      idx_v)
      for cp in copies(bufs):
        cp.start()

    def wait_chunk(bufs):
      for cp in copies(bufs):
        cp.wait()

    def tdot(xs, ys):
      if Q == 4:
        return (xs[0] * ys[0] + xs[1] * ys[1]) + (xs[2] * ys[2] + xs[3] * ys[3])
      return xs[0] * ys[0] + xs[1] * ys[1]

    def compute_chunk(bufs, ci):
      idx_v, a_v, r_v, v_v, _ = bufs
      lane = lax.iota(i32, L)

      def rsum(x):
        return jnp.broadcast_to(jnp.sum(x), (L,))

      def group_body(g, carry):
        score_acc = jnp.zeros((L,), f32)
        hoffv = idx_v[pl.ds(3 * _C + g * L, L)]
        toffv = idx_v[pl.ds(4 * _C + g * L, L)]
        roffv = idx_v[pl.ds(5 * _C + g * L, L)]
        for k in range(L):
          i = g * L + k
          hoff = hoffv[k]
          toff = toffv[k]
          roff = roffv[k]
          hs = [a_v[i, pl.ds(hoff + L * q, L)] for q in range(Q)]
          ts = [a_v[_C + i, pl.ds(toff + L * q, L)] for q in range(Q)]
          rs = [r_v[i, pl.ds(roff + L * q, L)] for q in range(Q)]
          vs = [v_v[i, pl.ds(roff + L * q, L)] for q in range(Q)]
          hv = rsum(tdot(hs, vs))
          tv = rsum(tdot(ts, vs))
          vv = rsum(tdot(vs, vs))
          c = (tv - hv) / jnp.maximum(vv, 1e-24)
          sq = [jnp.abs(hs[q] + rs[q] - ts[q] + c * vs[q]) for q in range(Q)]
          ssum = ((sq[0] + sq[1]) + (sq[2] + sq[3])) if Q == 4 else (
              sq[0] + sq[1])
          score = gamma - rsum(ssum)
          score_acc = jnp.where(lane == k, score, score_acc)

        sc_v[pl.ds(ci * _C + g * L, L)] = score_acc
        return carry

      lax.fori_loop(0, groups, group_body, 0)

    start_chunk(set0, 0)

    def chunk_pair(ci2, carry):
      ci = ci2 * 2
      wait_chunk(set0)
      start_chunk(set1, ci + 1)
      compute_chunk(set0, ci)
      wait_chunk(set1)

      @pl.when(ci + 2 < n_chunks)
      def _():
        start_chunk(set0, ci + 2)

      compute_chunk(set1, ci + 1)
      return carry

    lax.fori_loop(0, n_chunks // 2, chunk_pair, 0)
    pltpu.sync_copy(sc_v, out_hbm.at[pl.ds(base, per_w)])

  return kge


def _pack(cols, B):
  arr = jnp.stack(cols)
  return arr.reshape(_NIDX, B // _C, _C).transpose(1, 0, 2).reshape(-1)


def kernel(sample, entity_embedding, relation_embedding, type_embedding,
           reltype_embedding, norm_vector_embedding, norm_vectortype_embedding):
  B = sample.shape[0]
  E = entity_embedding.shape[0]
  R = relation_embedding.shape[0]
  E2, E4, R2, R4 = E // 2, E // 4, R // 2, R // 4
  assert E == R

  ent2, rel2, nv2 = _build_relayout(E, 3, _HID)(
      entity_embedding, relation_embedding, norm_vector_embedding)
  typ4, rtyp4, nvt4 = _build_relayout(E, 3, _TDIM)(
      type_embedding, reltype_embedding, norm_vectortype_embedding)

  h = sample[:, 0]
  r = sample[:, 1]
  t = sample[:, 2]

  def split(x, n, w):
    q = x // n
    return x - q * n, (q * w).astype(jnp.int32)

  h2, hoff = split(h, E2, _HID)
  t2, toff = split(t, E2, _HID)
  r2, roff = split(r, R2, _HID)
  h4, h4o = split(h, E4, _TDIM)
  t4, t4o = split(t, E4, _TDIM)
  r4, r4o = split(r, R4, _TDIM)

  pack_w = _pack([h2, t2, r2, hoff, toff, roff], B)
  pack_n = _pack([h4, t4, r4, h4o, t4o, r4o], B)

  score = _build_sc(B, _HID, _GAMMA)(pack_w, ent2, rel2, nv2)
  score_type = _build_sc(B, _TDIM, _GAMMA_TYPE)(pack_n, typ4, rtyp4, nvt4)
  return score.reshape(B, 1), score_type.reshape(B, 1)


# speedup vs baseline: 1.5162x; 1.5162x over previous
import functools

import jax
import jax.numpy as jnp
from jax import lax
from jax.experimental import pallas as pl
from jax.experimental.pallas import tpu as pltpu
from jax.experimental.pallas import tpu_sc as plsc

_GAMMA = 12.0
_GAMMA_TYPE = 6.0
_HID = 64
_TDIM = 32
_C = 64
_NIDX = 6
_WB = 400
_NB = 200


def _stack128(tbl):
  N, w = tbl.shape
  k = 128 // w
  n = N // k
  return jnp.concatenate([tbl[i * n:(i + 1) * n] for i in range(k)], axis=1)


@functools.cache
def _build_sc(B, D, gamma):
  info = plsc.get_sparse_core_info()
  NC, NS, L = info.num_cores, info.num_subcores, info.num_lanes
  NW = NC * NS
  Q = D // L
  assert B % (NW * _C) == 0
  per_w = B // NW
  n_chunks = per_w // _C
  assert n_chunks % 2 == 0
  groups = _C // L
  f32 = jnp.float32
  i32 = jnp.int32
  mesh = plsc.VectorSubcoreMesh(core_axis_name="c", subcore_axis_name="s")

  def _set():
    return [
        pltpu.VMEM((_NIDX * _C,), i32),
        pltpu.VMEM((2 * _C, 128), f32),
        pltpu.VMEM((_C, 128), f32),
        pltpu.VMEM((_C, 128), f32),
        pltpu.SemaphoreType.DMA,
    ]

  @functools.partial(
      pl.kernel,
      mesh=mesh,
      compiler_params=pltpu.CompilerParams(
          needs_layout_passes=False,
          disable_bounds_checks=True,
      ),
      out_type=jax.ShapeDtypeStruct((B,), f32),
      scratch_types=_set() + _set() + [pltpu.VMEM((per_w,), f32)],
  )
  def kge(pack_hbm, a_hbm, r_hbm, v_hbm, out_hbm, *scratch):
    set0 = scratch[0:5]
    set1 = scratch[5:10]
    sc_v = scratch[10]
    wid = lax.axis_index("s") * NC + lax.axis_index("c")
    base = wid * per_w

    def copies(bufs):
      idx_v, a_v, r_v, v_v, sem = bufs
      ht = idx_v.at[pl.ds(0, 2 * _C)]
      rr = idx_v.at[pl.ds(2 * _C, _C)]
      return [
          pltpu.make_async_copy(a_hbm.at[ht], a_v, sem),
          pltpu.make_async_copy(r_hbm.at[rr], r_v, sem),
          pltpu.make_async_copy(v_hbm.at[rr], v_v, sem),
      ]

    def start_chunk(bufs, ci):
      idx_v = bufs[0]
      gchunk = wid * n_chunks + ci
      pltpu.sync_copy(pack_hbm.at[pl.ds(gchunk * (_NIDX * _C), _NIDX * _C)],
                      idx_v)
      for cp in copies(bufs):
        cp.start()

    def wait_chunk(bufs):
      for cp in copies(bufs):
        cp.wait()

    def tdot(xs, ys):
      if Q == 4:
        return (xs[0] * ys[0] + xs[1] * ys[1]) + (xs[2] * ys[2] + xs[3] * ys[3])
      return xs[0] * ys[0] + xs[1] * ys[1]

    def compute_chunk(bufs, ci):
      idx_v, a_v, r_v, v_v, _ = bufs
      lane = lax.iota(i32, L)

      def rsum(x):
        return jnp.broadcast_to(jnp.sum(x), (L,))

      def group_body(g, carry):
        score_acc = jnp.zeros((L,), f32)
        hoffv = idx_v[pl.ds(3 * _C + g * L, L)]
        toffv = idx_v[pl.ds(4 * _C + g * L, L)]
        roffv = idx_v[pl.ds(5 * _C + g * L, L)]
        for k in range(L):
          i = g * L + k
          hoff = hoffv[k]
          toff = toffv[k]
          roff = roffv[k]
          hs = [a_v[i, pl.ds(hoff + L * q, L)] for q in range(Q)]
          ts = [a_v[_C + i, pl.ds(toff + L * q, L)] for q in range(Q)]
          rs = [r_v[i, pl.ds(roff + L * q, L)] for q in range(Q)]
          vs = [v_v[i, pl.ds(roff + L * q, L)] for q in range(Q)]
          hv = rsum(tdot(hs, vs))
          tv = rsum(tdot(ts, vs))
          vv = rsum(tdot(vs, vs))
          c = (tv - hv) / jnp.maximum(vv, 1e-24)
          sq = [jnp.abs(hs[q] + rs[q] - ts[q] + c * vs[q]) for q in range(Q)]
          ssum = ((sq[0] + sq[1]) + (sq[2] + sq[3])) if Q == 4 else (
              sq[0] + sq[1])
          score = gamma - rsum(ssum)
          score_acc = jnp.where(lane == k, score, score_acc)

        sc_v[pl.ds(ci * _C + g * L, L)] = score_acc
        return carry

      lax.fori_loop(0, groups, group_body, 0)

    start_chunk(set0, 0)

    def chunk_pair(ci2, carry):
      ci = ci2 * 2
      wait_chunk(set0)
      start_chunk(set1, ci + 1)
      compute_chunk(set0, ci)
      wait_chunk(set1)

      @pl.when(ci + 2 < n_chunks)
      def _():
        start_chunk(set0, ci + 2)

      compute_chunk(set1, ci + 1)
      return carry

    lax.fori_loop(0, n_chunks // 2, chunk_pair, 0)
    pltpu.sync_copy(sc_v, out_hbm.at[pl.ds(base, per_w)])

  return kge


def _pack(cols, B):
  arr = jnp.stack(cols)
  return arr.reshape(_NIDX, B // _C, _C).transpose(1, 0, 2).reshape(-1)


def kernel(sample, entity_embedding, relation_embedding, type_embedding,
           reltype_embedding, norm_vector_embedding, norm_vectortype_embedding):
  B = sample.shape[0]
  E = entity_embedding.shape[0]
  R = relation_embedding.shape[0]
  E2, E4, R2, R4 = E // 2, E // 4, R // 2, R // 4
  assert E == R

  ent2 = _stack128(entity_embedding)
  rel2 = _stack128(relation_embedding)
  nv2 = _stack128(norm_vector_embedding)
  typ4 = _stack128(type_embedding)
  rtyp4 = _stack128(reltype_embedding)
  nvt4 = _stack128(norm_vectortype_embedding)

  h = sample[:, 0]
  r = sample[:, 1]
  t = sample[:, 2]

  def split(x, n, w):
    q = x // n
    return x - q * n, (q * w).astype(jnp.int32)

  h2, hoff = split(h, E2, _HID)
  t2, toff = split(t, E2, _HID)
  r2, roff = split(r, R2, _HID)
  h4, h4o = split(h, E4, _TDIM)
  t4, t4o = split(t, E4, _TDIM)
  r4, r4o = split(r, R4, _TDIM)

  pack_w = _pack([h2, t2, r2, hoff, toff, roff], B)
  pack_n = _pack([h4, t4, r4, h4o, t4o, r4o], B)

  score = _build_sc(B, _HID, _GAMMA)(pack_w, ent2, rel2, nv2)
  score_type = _build_sc(B, _TDIM, _GAMMA_TYPE)(pack_n, typ4, rtyp4, nvt4)
  return score.reshape(B, 1), score_type.reshape(B, 1)


# speedup vs baseline: 2.2806x; 1.5041x over previous
import functools

import jax
import jax.numpy as jnp
from jax import lax
from jax.experimental import pallas as pl
from jax.experimental.pallas import tpu as pltpu
from jax.experimental.pallas import tpu_sc as plsc

_GAMMA = 12.0
_GAMMA_TYPE = 6.0
_HID = 64
_TDIM = 32
_C = 128
_NIDX = 3


@functools.cache
def _build(B):
  info = plsc.get_sparse_core_info()
  NC, NS, L = info.num_cores, info.num_subcores, info.num_lanes
  NW = NC * NS
  assert B % (NW * _C) == 0
  per_w = B // NW
  n_chunks = per_w // _C
  assert n_chunks % 2 == 0
  groups = _C // L
  f32 = jnp.float32
  i32 = jnp.int32
  mesh = plsc.VectorSubcoreMesh(core_axis_name="c", subcore_axis_name="s")

  def _set():
    return [
        pltpu.VMEM((_NIDX * _C,), i32),
        pltpu.VMEM((2 * _C, _HID), f32),
        pltpu.VMEM((2 * _C, _TDIM), f32),
        pltpu.VMEM((_C, _HID), f32),
        pltpu.VMEM((_C, _HID), f32),
        pltpu.VMEM((_C, _TDIM), f32),
        pltpu.VMEM((_C, _TDIM), f32),
        pltpu.SemaphoreType.DMA,
    ]

  @functools.partial(
      pl.kernel,
      mesh=mesh,
      compiler_params=pltpu.CompilerParams(
          needs_layout_passes=False,
          use_tc_tiling_on_sc=False,
          disable_bounds_checks=True,
      ),
      out_type=[jax.ShapeDtypeStruct((B,), f32),
                jax.ShapeDtypeStruct((B,), f32)],
      scratch_types=(
          _set() + _set()
          + [
              pltpu.VMEM((per_w,), f32),
              pltpu.VMEM((per_w,), f32),
          ]),
  )
  def kge(pack_hbm, ent_hbm, rel_hbm, typ_hbm, rtyp_hbm, nv_hbm, nvt_hbm,
          score_hbm, scoret_hbm, *scratch):
    set0 = scratch[0:8]
    set1 = scratch[8:16]
    sc_v, sct_v = scratch[16:18]
    wid = lax.axis_index("s") * NC + lax.axis_index("c")
    base = wid * per_w

    def copies(bufs):
      idx_v, ent_v, typ_v, rel_v, nv_v, rtyp_v, nvt_v, sem = bufs
      ht = idx_v.at[pl.ds(0, 2 * _C)]
      rr = idx_v.at[pl.ds(2 * _C, _C)]
      return [
          pltpu.make_async_copy(ent_hbm.at[ht], ent_v, sem),
          pltpu.make_async_copy(typ_hbm.at[ht], typ_v, sem),
          pltpu.make_async_copy(rel_hbm.at[rr], rel_v, sem),
          pltpu.make_async_copy(nv_hbm.at[rr], nv_v, sem),
          pltpu.make_async_copy(rtyp_hbm.at[rr], rtyp_v, sem),
          pltpu.make_async_copy(nvt_hbm.at[rr], nvt_v, sem),
      ]

    def start_chunk(bufs, ci):
      idx_v = bufs[0]
      gchunk = wid * n_chunks + ci
      pltpu.sync_copy(pack_hbm.at[pl.ds(gchunk * (_NIDX * _C), _NIDX * _C)],
                      idx_v)
      for cp in copies(bufs):
        cp.start()

    def wait_chunk(bufs):
      for cp in copies(bufs):
        cp.wait()

    def compute_chunk(bufs, ci):
      idx_v, ent_v, typ_v, rel_v, nv_v, rtyp_v, nvt_v, _ = bufs
      lane = lax.iota(i32, L)

      def rsum(x):
        return jnp.broadcast_to(jnp.sum(x), (L,))

      def group_body(g, carry):
        score_acc = jnp.zeros((L,), f32)
        scoret_acc = jnp.zeros((L,), f32)
        for k in range(L):
          i = g * L + k

          hs = [ent_v[i, pl.ds(16 * q, 16)] for q in range(4)]
          ts = [ent_v[_C + i, pl.ds(16 * q, 16)] for q in range(4)]
          rs = [rel_v[i, pl.ds(16 * q, 16)] for q in range(4)]
          vs = [nv_v[i, pl.ds(16 * q, 16)] for q in range(4)]
          hv = rsum((hs[0] * vs[0] + hs[1] * vs[1])
                    + (hs[2] * vs[2] + hs[3] * vs[3]))
          tv = rsum((ts[0] * vs[0] + ts[1] * vs[1])
                    + (ts[2] * vs[2] + ts[3] * vs[3]))
          vv = rsum((vs[0] * vs[0] + vs[1] * vs[1])
                    + (vs[2] * vs[2] + vs[3] * vs[3]))
          c = (tv - hv) / jnp.maximum(vv, 1e-24)
          s4 = [jnp.abs(hs[q] + rs[q] - ts[q] + c * vs[q]) for q in range(4)]
          score = _GAMMA - rsum((s4[0] + s4[1]) + (s4[2] + s4[3]))

          h2s = [typ_v[i, pl.ds(16 * q, 16)] for q in range(2)]
          t2s = [typ_v[_C + i, pl.ds(16 * q, 16)] for q in range(2)]
          r2s = [rtyp_v[i, pl.ds(16 * q, 16)] for q in range(2)]
          v2s = [nvt_v[i, pl.ds(16 * q, 16)] for q in range(2)]
          hv2 = rsum(h2s[0] * v2s[0] + h2s[1] * v2s[1])
          tv2 = rsum(t2s[0] * v2s[0] + t2s[1] * v2s[1])
          vv2 = rsum(v2s[0] * v2s[0] + v2s[1] * v2s[1])
          c2 = (tv2 - hv2) / jnp.maximum(vv2, 1e-24)
          s2 = [jnp.abs(h2s[q] + r2s[q] - t2s[q] + c2 * v2s[q])
                for q in range(2)]
          score_t = _GAMMA_TYPE - rsum(s2[0] + s2[1])

          score_acc = jnp.where(lane == k, score, score_acc)
          scoret_acc = jnp.where(lane == k, score_t, scoret_acc)

        out_off = ci * _C + g * L
        sc_v[pl.ds(out_off, L)] = score_acc
        sct_v[pl.ds(out_off, L)] = scoret_acc
        return carry

      lax.fori_loop(0, groups, group_body, 0)

    start_chunk(set0, 0)

    def chunk_pair(ci2, carry):
      ci = ci2 * 2
      wait_chunk(set0)
      start_chunk(set1, ci + 1)
      compute_chunk(set0, ci)
      wait_chunk(set1)

      @pl.when(ci + 2 < n_chunks)
      def _():
        start_chunk(set0, ci + 2)

      compute_chunk(set1, ci + 1)
      return carry

    lax.fori_loop(0, n_chunks // 2, chunk_pair, 0)
    pltpu.sync_copy(sc_v, score_hbm.at[pl.ds(base, per_w)])
    pltpu.sync_copy(sct_v, scoret_hbm.at[pl.ds(base, per_w)])

  return kge


def kernel(sample, entity_embedding, relation_embedding, type_embedding,
           reltype_embedding, norm_vector_embedding, norm_vectortype_embedding):
  B = sample.shape[0]
  fn = _build(B)
  h = sample[:, 0]
  r = sample[:, 1]
  t = sample[:, 2]
  idx3 = jnp.stack([h, t, r])
  pack = idx3.reshape(_NIDX, B // _C, _C).transpose(1, 0, 2).reshape(-1)
  score, score_type = fn(
      pack, entity_embedding, relation_embedding, type_embedding,
      reltype_embedding, norm_vector_embedding, norm_vectortype_embedding)
  return score.reshape(B, 1), score_type.reshape(B, 1)
